# initial kernel scaffold (unmeasured)
import jax
import jax.numpy as jnp
from jax import lax
from jax.experimental import pallas as pl
from jax.experimental.pallas import tpu as pltpu


def kernel(
    x,
):
    def body(*refs):
        pass

    out_shape = jax.ShapeDtypeStruct(..., jnp.float32)
    return pl.pallas_call(body, out_shape=out_shape)(...)


# baseline (device time: 31816 ns/iter reference)
import jax
import jax.numpy as jnp
from jax import lax
from jax.experimental import pallas as pl
from jax.experimental.pallas import tpu as pltpu

M = 2048
N = 1024
N_HALF = N // 2


def kernel(x):
    def body(x_ref, out_ref, send_ref, recv_ref, send_sem, recv_sem):
        my_x = lax.axis_index("x")
        my_y = lax.axis_index("y")
        my_z = lax.axis_index("z")
        peer_y = 1 - my_y

        barrier_sem = pltpu.get_barrier_semaphore()
        pl.semaphore_signal(
            barrier_sem,
            inc=1,
            device_id=(my_x, peer_y, my_z),
            device_id_type=pl.DeviceIdType.MESH,
        )
        pl.semaphore_wait(barrier_sem, 1)

        send_ref[...] = x_ref[0, :, pl.ds(peer_y * N_HALF, N_HALF)].astype(
            jnp.bfloat16
        )

        rdma = pltpu.make_async_remote_copy(
            src_ref=send_ref,
            dst_ref=recv_ref,
            send_sem=send_sem,
            recv_sem=recv_sem,
            device_id=(my_x, peer_y, my_z),
            device_id_type=pl.DeviceIdType.MESH,
        )
        rdma.start()
        rdma.wait()

        out_ref[...] = x_ref[0, :, pl.ds(my_y * N_HALF, N_HALF)] + recv_ref[
            ...
        ].astype(jnp.float32)

    return pl.pallas_call(
        body,
        out_shape=jax.ShapeDtypeStruct((M, N_HALF), jnp.float32),
        in_specs=[pl.BlockSpec(memory_space=pltpu.VMEM)],
        out_specs=pl.BlockSpec(memory_space=pltpu.VMEM),
        scratch_shapes=[
            pltpu.VMEM((M, N_HALF), jnp.bfloat16),
            pltpu.VMEM((M, N_HALF), jnp.bfloat16),
            pltpu.SemaphoreType.DMA,
            pltpu.SemaphoreType.DMA,
        ],
        compiler_params=pltpu.CompilerParams(collective_id=0),
    )(x)


# device time: 24257 ns/iter; 1.3116x vs baseline; 1.3116x over previous
import jax
import jax.numpy as jnp
from jax import lax
from jax.experimental import pallas as pl
from jax.experimental.pallas import tpu as pltpu

M = 2048
N = 1024
N_HALF = N // 2
M_HALF = M // 2
C = 8
CK = M_HALF // C


def kernel(x):
    def body(
        x_ref,
        out_ref,
        y_send,
        y_recv,
        x_recv,
        y_send_sems,
        y_recv_sems,
        x_send_sems,
        x_recv_sems,
    ):
        my_x = lax.axis_index("x")
        my_y = lax.axis_index("y")
        my_z = lax.axis_index("z")
        peer_y = 1 - my_y
        peer_x = 1 - my_x

        my_rows = my_x * M_HALF
        other_rows = peer_x * M_HALF
        my_cols = my_y * N_HALF
        send_cols = peer_y * N_HALF

        barrier_sem = pltpu.get_barrier_semaphore()
        for dev in ((my_x, peer_y, my_z), (peer_x, my_y, my_z)):
            pl.semaphore_signal(
                barrier_sem,
                inc=1,
                device_id=dev,
                device_id_type=pl.DeviceIdType.MESH,
            )
        pl.semaphore_wait(barrier_sem, 2)

        y_dmas = []
        for i in range(C):
            y_send[pl.ds(i * CK, CK)] = x_ref[
                0, pl.ds(my_rows + i * CK, CK), pl.ds(send_cols, N_HALF)
            ].astype(jnp.bfloat16)
            d = pltpu.make_async_remote_copy(
                src_ref=y_send.at[pl.ds(i * CK, CK)],
                dst_ref=y_recv.at[pl.ds(i * CK, CK)],
                send_sem=y_send_sems.at[i],
                recv_sem=y_recv_sems.at[i],
                device_id=(my_x, peer_y, my_z),
                device_id_type=pl.DeviceIdType.MESH,
            )
            d.start()
            y_dmas.append(d)

        x_dmas = []
        for i in range(C):
            y_dmas[i].wait_recv()
            d = pltpu.make_async_remote_copy(
                src_ref=y_recv.at[pl.ds(i * CK, CK)],
                dst_ref=x_recv.at[pl.ds(i * CK, CK)],
                send_sem=x_send_sems.at[i],
                recv_sem=x_recv_sems.at[i],
                device_id=(peer_x, my_y, my_z),
                device_id_type=pl.DeviceIdType.MESH,
            )
            d.start()
            x_dmas.append(d)
            out_ref[pl.ds(my_rows + i * CK, CK), :] = x_ref[
                0, pl.ds(my_rows + i * CK, CK), pl.ds(my_cols, N_HALF)
            ] + y_recv[pl.ds(i * CK, CK)].astype(jnp.float32)

        for i in range(C):
            x_dmas[i].wait_recv()
            out_ref[pl.ds(other_rows + i * CK, CK), :] = x_ref[
                0, pl.ds(other_rows + i * CK, CK), pl.ds(my_cols, N_HALF)
            ] + x_recv[pl.ds(i * CK, CK)].astype(jnp.float32)

        for i in range(C):
            y_dmas[i].wait_send()
            x_dmas[i].wait_send()

    return pl.pallas_call(
        body,
        out_shape=jax.ShapeDtypeStruct((M, N_HALF), jnp.float32),
        in_specs=[pl.BlockSpec(memory_space=pltpu.VMEM)],
        out_specs=pl.BlockSpec(memory_space=pltpu.VMEM),
        scratch_shapes=[
            pltpu.VMEM((M_HALF, N_HALF), jnp.bfloat16),
            pltpu.VMEM((M_HALF, N_HALF), jnp.bfloat16),
            pltpu.VMEM((M_HALF, N_HALF), jnp.bfloat16),
            pltpu.SemaphoreType.DMA((C,)),
            pltpu.SemaphoreType.DMA((C,)),
            pltpu.SemaphoreType.DMA((C,)),
            pltpu.SemaphoreType.DMA((C,)),
        ],
        compiler_params=pltpu.CompilerParams(collective_id=0),
    )(x)
